# Initial kernel scaffold; baseline (speedup 1.0000x reference)
import functools

import jax
import jax.numpy as jnp
from jax import lax
from jax.experimental import pallas as pl
from jax.experimental.pallas import tpu as pltpu
from jax.experimental.pallas import tpu_sc as plsc

N = 10000
E = 320000
H = 128
G = 64
OUT = 12
EPS = 1e-5

NC = 2
NS = 16
NW = NC * NS
CH = 128
NCHUNK = E // CH
BASE_CH = NCHUNK // NW
EXTRA_CH = NCHUNK % NW
ZROWS = 125
DEGW = 16

_HIGH = lax.Precision.HIGHEST

_sc_mesh = plsc.VectorSubcoreMesh(core_axis_name="c", subcore_axis_name="s")


def _deg_body(dst_hbm, out_hbm, dstv, onesv, zbuf, acc):
    cid = lax.axis_index("c")
    sid = lax.axis_index("s")
    wid = sid * NC + cid

    @pl.loop(0, CH)
    def _(i):
        onesv[i, :] = jnp.ones((DEGW,), jnp.float32)

    @pl.loop(0, ZROWS)
    def _(i):
        zbuf[i, :] = jnp.zeros((DEGW,), jnp.float32)

    @pl.loop(0, 5)
    def _(k):
        pltpu.sync_copy(zbuf, acc.at[pl.ds(sid * 625 + k * ZROWS, ZROWS)])

    plsc.subcore_barrier()

    nch = jnp.where(wid < EXTRA_CH, BASE_CH + 1, BASE_CH)

    @pl.loop(0, nch)
    def _(c):
        off = (c * NW + wid) * CH
        pltpu.sync_copy(dst_hbm.at[pl.ds(off, CH)], dstv)
        pltpu.sync_copy(onesv, acc.at[dstv], add=True)

    plsc.subcore_barrier()

    @pl.loop(0, 5)
    def _(k):
        r0 = sid * 625 + k * ZROWS
        pltpu.sync_copy(acc.at[pl.ds(r0, ZROWS)],
                        out_hbm.at[pl.ds(cid * N + r0, ZROWS)])


@jax.jit
def _sc_deg(dst):
    k = pl.kernel(
        _deg_body,
        out_type=jax.ShapeDtypeStruct((NC * N, DEGW), jnp.float32),
        mesh=_sc_mesh,
        scratch_types=[
            pltpu.VMEM((CH,), jnp.int32),
            pltpu.VMEM((CH, DEGW), jnp.float32),
            pltpu.VMEM((ZROWS, DEGW), jnp.float32),
            pltpu.VMEM_SHARED((N, DEGW), jnp.float32),
        ],
    )
    return k(dst)


def _agg_body(hs_hbm, src_hbm, dst_hbm, out_hbm,
              srcv, dstv, rows, zbuf, acc, gsem):
    cid = lax.axis_index("c")
    sid = lax.axis_index("s")
    wid = sid * NC + cid

    @pl.loop(0, ZROWS)
    def _(i):
        @pl.loop(0, H // 16)
        def _(j):
            zbuf[i, pl.ds(j * 16, 16)] = jnp.zeros((16,), jnp.float32)

    @pl.loop(0, 5)
    def _(k):
        pltpu.sync_copy(zbuf, acc.at[pl.ds(sid * 625 + k * ZROWS, ZROWS)])

    plsc.subcore_barrier()

    nch = jnp.where(wid < EXTRA_CH, BASE_CH + 1, BASE_CH)

    @pl.loop(0, nch)
    def _(c):
        off = (c * NW + wid) * CH
        pltpu.sync_copy(src_hbm.at[pl.ds(off, CH)], srcv)
        pltpu.sync_copy(dst_hbm.at[pl.ds(off, CH)], dstv)
        pltpu.async_copy(hs_hbm.at[srcv], rows, gsem).wait()
        pltpu.sync_copy(rows, acc.at[dstv], add=True)

    plsc.subcore_barrier()

    @pl.loop(0, 5)
    def _(k):
        r0 = sid * 625 + k * ZROWS
        pltpu.sync_copy(acc.at[pl.ds(r0, ZROWS)],
                        out_hbm.at[pl.ds(cid * N + r0, ZROWS)])


@jax.jit
def _sc_agg(hs, src, dst):
    k = pl.kernel(
        _agg_body,
        out_type=jax.ShapeDtypeStruct((NC * N, H), jnp.float32),
        mesh=_sc_mesh,
        scratch_types=[
            pltpu.VMEM((CH,), jnp.int32),
            pltpu.VMEM((CH,), jnp.int32),
            pltpu.VMEM((CH, H), jnp.float32),
            pltpu.VMEM((ZROWS, H), jnp.float32),
            pltpu.VMEM_SHARED((N, H), jnp.float32),
            pltpu.SemaphoreType.DMA,
        ],
    )
    return k(hs, src, dst)


def _emb_kernel(x_ref, wemb_ref, bemb_ref, wc_ref, o_ref):
    h = jnp.dot(x_ref[...], wemb_ref[...], precision=_HIGH,
                preferred_element_type=jnp.float32) + bemb_ref[...]
    o_ref[...] = jnp.dot(h, wc_ref[...], precision=_HIGH,
                         preferred_element_type=jnp.float32)


def _tc_emb(x, W_emb, b_emb, W_c0):
    return pl.pallas_call(
        _emb_kernel,
        out_shape=jax.ShapeDtypeStruct((N, H), jnp.float32),
    )(x, W_emb, b_emb.reshape(1, H), W_c0)


def _scale_kernel(degp_ref, hl_ref, hs_ref, dinv_ref):
    deg = degp_ref[0:N, 0:1] + degp_ref[N:2 * N, 0:1] + 1.0
    dinv = lax.rsqrt(deg)
    dinv_ref[...] = dinv
    hs_ref[...] = hl_ref[...] * dinv


def _tc_scale(degp, hl):
    return pl.pallas_call(
        _scale_kernel,
        out_shape=(jax.ShapeDtypeStruct((N, H), jnp.float32),
                   jax.ShapeDtypeStruct((N, 1), jnp.float32)),
    )(degp, hl)


def _mid_kernel(accp_ref, hs_ref, dinv_ref, b_ref, g_ref, bb_ref, w_ref,
                o_ref):
    dinv = dinv_ref[...]
    pre = (accp_ref[0:N, :] + accp_ref[N:2 * N, :] + hs_ref[...]) * dinv \
        + b_ref[...]
    mean = jnp.mean(pre, axis=0, keepdims=True)
    var = jnp.mean((pre - mean) ** 2, axis=0, keepdims=True)
    hn = (pre - mean) * lax.rsqrt(var + EPS) * g_ref[...] + bb_ref[...]
    h = jnp.maximum(hn, 0.0)
    o_ref[...] = jnp.dot(h, w_ref[...], precision=_HIGH,
                         preferred_element_type=jnp.float32) * dinv


def _tc_mid(accp, hs, dinv, b, g, bb, W_next):
    return pl.pallas_call(
        _mid_kernel,
        out_shape=jax.ShapeDtypeStruct((N, H), jnp.float32),
    )(accp, hs, dinv, b.reshape(1, H), g.reshape(1, H), bb.reshape(1, H),
      W_next)


def _final_kernel(accp_ref, hs_ref, dinv_ref, b_ref, g_ref, bb_ref,
                  batch_ref, wh1_ref, bh1_ref, wh2_ref, bh2_ref, o_ref):
    dinv = dinv_ref[...]
    pre = (accp_ref[0:N, :] + accp_ref[N:2 * N, :] + hs_ref[...]) * dinv \
        + b_ref[...]
    mean = jnp.mean(pre, axis=0, keepdims=True)
    var = jnp.mean((pre - mean) ** 2, axis=0, keepdims=True)
    hn = (pre - mean) * lax.rsqrt(var + EPS) * g_ref[...] + bb_ref[...]
    h = jnp.maximum(hn, 0.0)
    seg = lax.broadcasted_iota(jnp.int32, (N, G), 1)
    onehot = (batch_ref[...] == seg).astype(jnp.float32)
    pooled = lax.dot_general(onehot, h, (((0,), (0,)), ((), ())),
                             precision=_HIGH,
                             preferred_element_type=jnp.float32)
    z = jnp.maximum(
        jnp.dot(pooled, wh1_ref[...], precision=_HIGH,
                preferred_element_type=jnp.float32) + bh1_ref[...], 0.0)
    o_ref[...] = jnp.dot(z, wh2_ref[...], precision=_HIGH,
                         preferred_element_type=jnp.float32) + bh2_ref[...]


def _tc_final(accp, hs, dinv, b, g, bb, batch2d, W_h1, b_h1, W_h2, b_h2):
    return pl.pallas_call(
        _final_kernel,
        out_shape=jax.ShapeDtypeStruct((G, OUT), jnp.float32),
    )(accp, hs, dinv, b.reshape(1, H), g.reshape(1, H), bb.reshape(1, H),
      batch2d, W_h1, b_h1.reshape(1, H // 2), W_h2, b_h2.reshape(1, OUT))


def kernel(x, edge_index, batch, W_emb, b_emb, W_c0, b_c0, bn_g0, bn_b0,
           W_c1, b_c1, bn_g1, bn_b1, W_c2, b_c2, bn_g2, bn_b2,
           W_h1, b_h1, W_h2, b_h2):
    src = edge_index[0]
    dst = edge_index[1]
    batch2d = batch.reshape(N, 1)

    degp = _sc_deg(dst)
    hl0 = _tc_emb(x, W_emb, b_emb, W_c0)
    hs0, dinv = _tc_scale(degp, hl0)

    acc0 = _sc_agg(hs0, src, dst)
    hs1 = _tc_mid(acc0, hs0, dinv, b_c0, bn_g0, bn_b0, W_c1)
    acc1 = _sc_agg(hs1, src, dst)
    hs2 = _tc_mid(acc1, hs1, dinv, b_c1, bn_g1, bn_b1, W_c2)
    acc2 = _sc_agg(hs2, src, dst)

    return _tc_final(acc2, hs2, dinv, b_c2, bn_g2, bn_b2, batch2d,
                     W_h1, b_h1, W_h2, b_h2)


# speedup vs baseline: 13.9390x; 13.9390x over previous
import functools

import jax
import jax.numpy as jnp
from jax import lax
from jax.experimental import pallas as pl
from jax.experimental.pallas import tpu as pltpu
from jax.experimental.pallas import tpu_sc as plsc

N = 10000
E = 320000
H = 128
G = 64
OUT = 12
EPS = 1e-5

NC = 2
NS = 16
NW = NC * NS
CH = 128
NCHUNK = E // CH
BASE_CH = NCHUNK // NW
EXTRA_CH = NCHUNK % NW
ZROWS = 80
NQ = N // ZROWS
DEGW = 16

_HIGH = lax.Precision.HIGHEST

@functools.lru_cache(maxsize=None)
def _sc_mesh():
    return plsc.VectorSubcoreMesh(core_axis_name="c", subcore_axis_name="s",
                                  num_cores=NC)


def _deg_body(dst_hbm, out_hbm, dstv, onesv, zbuf, acc):
    cid = lax.axis_index("c")
    sid = lax.axis_index("s")
    wid = sid * NC + cid

    @pl.loop(0, CH)
    def _(i):
        onesv[i, :] = jnp.ones((DEGW,), jnp.float32)

    @pl.loop(0, ZROWS)
    def _(i):
        zbuf[i, :] = jnp.zeros((DEGW,), jnp.float32)

    nq = jnp.where(sid < NQ % NS, NQ // NS + 1, NQ // NS)

    @pl.loop(0, nq)
    def _(k):
        pltpu.sync_copy(zbuf, acc.at[pl.ds((k * NS + sid) * ZROWS, ZROWS)])

    plsc.subcore_barrier()

    nch = jnp.where(wid < EXTRA_CH, BASE_CH + 1, BASE_CH)

    @pl.loop(0, nch)
    def _(c):
        off = (c * NW + wid) * CH
        pltpu.sync_copy(dst_hbm.at[pl.ds(off, CH)], dstv)
        pltpu.sync_copy(onesv, acc.at[dstv], add=True)

    plsc.subcore_barrier()

    @pl.loop(0, nq)
    def _(k):
        r0 = (k * NS + sid) * ZROWS
        pltpu.sync_copy(acc.at[pl.ds(r0, ZROWS)],
                        out_hbm.at[pl.ds(cid * N + r0, ZROWS)])


@jax.jit
def _sc_deg(dst):
    k = pl.kernel(
        _deg_body,
        out_type=jax.ShapeDtypeStruct((NC * N, DEGW), jnp.float32),
        mesh=_sc_mesh(),
        scratch_types=[
            pltpu.VMEM((CH,), jnp.int32),
            pltpu.VMEM((CH, DEGW), jnp.float32),
            pltpu.VMEM((ZROWS, DEGW), jnp.float32),
            pltpu.VMEM_SHARED((N, DEGW), jnp.float32),
        ],
    )
    return k(dst)


def _agg_body(hs_hbm, src_hbm, dst_hbm, out_hbm,
              srcv, dstv, rows, zbuf, acc, gsem):
    cid = lax.axis_index("c")
    sid = lax.axis_index("s")
    wid = sid * NC + cid

    @pl.loop(0, ZROWS)
    def _(i):
        @pl.loop(0, H // 16)
        def _(j):
            zbuf[i, pl.ds(j * 16, 16)] = jnp.zeros((16,), jnp.float32)

    nq = jnp.where(sid < NQ % NS, NQ // NS + 1, NQ // NS)

    @pl.loop(0, nq)
    def _(k):
        pltpu.sync_copy(zbuf, acc.at[pl.ds((k * NS + sid) * ZROWS, ZROWS)])

    plsc.subcore_barrier()

    nch = jnp.where(wid < EXTRA_CH, BASE_CH + 1, BASE_CH)

    @pl.loop(0, nch)
    def _(c):
        off = (c * NW + wid) * CH
        pltpu.sync_copy(src_hbm.at[pl.ds(off, CH)], srcv)
        pltpu.sync_copy(dst_hbm.at[pl.ds(off, CH)], dstv)
        pltpu.async_copy(hs_hbm.at[srcv], rows, gsem).wait()
        pltpu.sync_copy(rows, acc.at[dstv], add=True)

    plsc.subcore_barrier()

    @pl.loop(0, nq)
    def _(k):
        r0 = (k * NS + sid) * ZROWS
        pltpu.sync_copy(acc.at[pl.ds(r0, ZROWS)],
                        out_hbm.at[pl.ds(cid * N + r0, ZROWS)])


@jax.jit
def _sc_agg(hs, src, dst):
    k = pl.kernel(
        _agg_body,
        out_type=jax.ShapeDtypeStruct((NC * N, H), jnp.float32),
        mesh=_sc_mesh(),
        scratch_types=[
            pltpu.VMEM((CH,), jnp.int32),
            pltpu.VMEM((CH,), jnp.int32),
            pltpu.VMEM((CH, H), jnp.float32),
            pltpu.VMEM((ZROWS, H), jnp.float32),
            pltpu.VMEM_SHARED((N, H), jnp.float32),
            pltpu.SemaphoreType.DMA,
        ],
    )
    return k(hs, src, dst)


def _emb_kernel(x_ref, wemb_ref, bemb_ref, wc_ref, o_ref):
    h = jnp.dot(x_ref[...], wemb_ref[...], precision=_HIGH,
                preferred_element_type=jnp.float32) + bemb_ref[...]
    o_ref[...] = jnp.dot(h, wc_ref[...], precision=_HIGH,
                         preferred_element_type=jnp.float32)


def _tc_emb(x, W_emb, b_emb, W_c0):
    return pl.pallas_call(
        _emb_kernel,
        out_shape=jax.ShapeDtypeStruct((N, H), jnp.float32),
    )(x, W_emb, b_emb.reshape(1, H), W_c0)


def _scale_kernel(degp_ref, hl_ref, hs_ref, dinv_ref):
    deg = degp_ref[0:N, 0:1] + degp_ref[N:2 * N, 0:1] + 1.0
    dinv = lax.rsqrt(deg)
    dinv_ref[...] = dinv
    hs_ref[...] = hl_ref[...] * dinv


def _tc_scale(degp, hl):
    return pl.pallas_call(
        _scale_kernel,
        out_shape=(jax.ShapeDtypeStruct((N, H), jnp.float32),
                   jax.ShapeDtypeStruct((N, 1), jnp.float32)),
    )(degp, hl)


def _mid_kernel(accp_ref, hs_ref, dinv_ref, b_ref, g_ref, bb_ref, w_ref,
                o_ref):
    dinv = dinv_ref[...]
    pre = (accp_ref[0:N, :] + accp_ref[N:2 * N, :] + hs_ref[...]) * dinv \
        + b_ref[...]
    mean = jnp.mean(pre, axis=0, keepdims=True)
    var = jnp.mean((pre - mean) ** 2, axis=0, keepdims=True)
    hn = (pre - mean) * lax.rsqrt(var + EPS) * g_ref[...] + bb_ref[...]
    h = jnp.maximum(hn, 0.0)
    o_ref[...] = jnp.dot(h, w_ref[...], precision=_HIGH,
                         preferred_element_type=jnp.float32) * dinv


def _tc_mid(accp, hs, dinv, b, g, bb, W_next):
    return pl.pallas_call(
        _mid_kernel,
        out_shape=jax.ShapeDtypeStruct((N, H), jnp.float32),
    )(accp, hs, dinv, b.reshape(1, H), g.reshape(1, H), bb.reshape(1, H),
      W_next)


def _final_kernel(accp_ref, hs_ref, dinv_ref, b_ref, g_ref, bb_ref,
                  batch_ref, wh1_ref, bh1_ref, wh2_ref, bh2_ref, o_ref):
    dinv = dinv_ref[...]
    pre = (accp_ref[0:N, :] + accp_ref[N:2 * N, :] + hs_ref[...]) * dinv \
        + b_ref[...]
    mean = jnp.mean(pre, axis=0, keepdims=True)
    var = jnp.mean((pre - mean) ** 2, axis=0, keepdims=True)
    hn = (pre - mean) * lax.rsqrt(var + EPS) * g_ref[...] + bb_ref[...]
    h = jnp.maximum(hn, 0.0)
    seg = lax.broadcasted_iota(jnp.int32, (N, G), 1)
    onehot = (batch_ref[...] == seg).astype(jnp.float32)
    pooled = lax.dot_general(onehot, h, (((0,), (0,)), ((), ())),
                             precision=_HIGH,
                             preferred_element_type=jnp.float32)
    z = jnp.maximum(
        jnp.dot(pooled, wh1_ref[...], precision=_HIGH,
                preferred_element_type=jnp.float32) + bh1_ref[...], 0.0)
    o_ref[...] = jnp.dot(z, wh2_ref[...], precision=_HIGH,
                         preferred_element_type=jnp.float32) + bh2_ref[...]


def _tc_final(accp, hs, dinv, b, g, bb, batch2d, W_h1, b_h1, W_h2, b_h2):
    return pl.pallas_call(
        _final_kernel,
        out_shape=jax.ShapeDtypeStruct((G, OUT), jnp.float32),
    )(accp, hs, dinv, b.reshape(1, H), g.reshape(1, H), bb.reshape(1, H),
      batch2d, W_h1, b_h1.reshape(1, H // 2), W_h2, b_h2.reshape(1, OUT))


def kernel(x, edge_index, batch, W_emb, b_emb, W_c0, b_c0, bn_g0, bn_b0,
           W_c1, b_c1, bn_g1, bn_b1, W_c2, b_c2, bn_g2, bn_b2,
           W_h1, b_h1, W_h2, b_h2):
    src = edge_index[0]
    dst = edge_index[1]
    batch2d = batch.reshape(N, 1)

    degp = _sc_deg(dst)
    hl0 = _tc_emb(x, W_emb, b_emb, W_c0)
    hs0, dinv = _tc_scale(degp, hl0)

    acc0 = _sc_agg(hs0, src, dst)
    hs1 = _tc_mid(acc0, hs0, dinv, b_c0, bn_g0, bn_b0, W_c1)
    acc1 = _sc_agg(hs1, src, dst)
    hs2 = _tc_mid(acc1, hs1, dinv, b_c1, bn_g1, bn_b1, W_c2)
    acc2 = _sc_agg(hs2, src, dst)

    return _tc_final(acc2, hs2, dinv, b_c2, bn_g2, bn_b2, batch2d,
                     W_h1, b_h1, W_h2, b_h2)


# speedup vs baseline: 26.5969x; 1.9081x over previous
import functools

import jax
import jax.numpy as jnp
from jax import lax
from jax.experimental import pallas as pl
from jax.experimental.pallas import tpu as pltpu
from jax.experimental.pallas import tpu_sc as plsc

N = 10000
E = 320000
H = 128
G = 64
OUT = 12
EPS = 1e-5

NC = 2
NS = 16
NW = NC * NS
CH = 128
NCHUNK = 2560
EPAD = NCHUNK * CH
REAL_NCHUNK = E // CH
TCH = NCHUNK // NW
NBUF = 2
THALF = 40
ZR = 8
NZ = 10000 // ZR
NXT = N + 16
ZROWS = 80
NQ = N // ZROWS
DEGW = 16

_HIGH = lax.Precision.HIGHEST

@functools.lru_cache(maxsize=None)
def _sc_mesh():
    return plsc.VectorSubcoreMesh(core_axis_name="c", subcore_axis_name="s",
                                  num_cores=NC)


def _deg_body(dst_hbm, out_hbm, dstidx, onesv, zbuf, acc, dsem, osem):
    cid = lax.axis_index("c")
    sid = lax.axis_index("s")
    wid = sid * NC + cid

    pltpu.async_copy(dst_hbm.at[pl.ds(wid * TCH, TCH)], dstidx, dsem)

    @pl.loop(0, CH)
    def _(i):
        onesv[i, :] = jnp.ones((DEGW,), jnp.float32)

    @pl.loop(0, ZROWS)
    def _(i):
        zbuf[i, :] = jnp.zeros((DEGW,), jnp.float32)

    nq = jnp.where(sid < NQ % NS, NQ // NS + 1, NQ // NS)

    @pl.loop(0, nq)
    def _(k):
        pltpu.sync_copy(zbuf, acc.at[pl.ds((k * NS + sid) * ZROWS, ZROWS)])

    pltpu.make_async_copy(dst_hbm.at[pl.ds(0, TCH)], dstidx, dsem).wait()
    plsc.subcore_barrier()

    nch = jnp.clip(REAL_NCHUNK - wid * TCH, 0, TCH)

    del osem

    @pl.loop(0, nch)
    def _(c):
        pltpu.sync_copy(onesv, acc.at[dstidx.at[c]], add=True)

    plsc.subcore_barrier()

    @pl.loop(0, nq)
    def _(k):
        r0 = (k * NS + sid) * ZROWS
        pltpu.sync_copy(acc.at[pl.ds(r0, ZROWS)],
                        out_hbm.at[pl.ds(cid * N + r0, ZROWS)])


@jax.jit
def _sc_deg(dst2d):
    k = pl.kernel(
        _deg_body,
        out_type=jax.ShapeDtypeStruct((NC * N, DEGW), jnp.float32),
        mesh=_sc_mesh(),
        scratch_types=[
            pltpu.VMEM((TCH, CH), jnp.int32),
            pltpu.VMEM((CH, DEGW), jnp.float32),
            pltpu.VMEM((ZROWS, DEGW), jnp.float32),
            pltpu.VMEM_SHARED((N, DEGW), jnp.float32),
            pltpu.SemaphoreType.DMA,
            pltpu.SemaphoreType.DMA,
        ],
    )
    return k(dst2d)


def _agg_body(hs_hbm, src_hbm, dst_hbm, out_hbm,
              srcidx, dstidx, rows0, rows1, zbuf, acc,
              isem, gsem0, gsem1):
    cid = lax.axis_index("c")
    sid = lax.axis_index("s")
    wid = sid * NC + cid
    rows = (rows0, rows1)
    gsem = (gsem0, gsem1)

    pltpu.async_copy(src_hbm.at[pl.ds(wid * TCH, THALF)], srcidx, isem)
    pltpu.async_copy(dst_hbm.at[pl.ds(wid * TCH, THALF)], dstidx, isem)

    @pl.loop(0, ZR)
    def _(i):
        @pl.loop(0, H // 16)
        def _(j):
            zbuf[i, pl.ds(j * 16, 16)] = jnp.zeros((16,), jnp.float32)

    nz = jnp.where(sid < NZ % NS, NZ // NS + 1, NZ // NS)

    @pl.loop(0, nz)
    def _(k):
        pltpu.sync_copy(zbuf, acc.at[pl.ds((k * NS + sid) * ZR, ZR)])

    pltpu.make_async_copy(src_hbm.at[pl.ds(0, THALF)], srcidx, isem).wait()
    pltpu.make_async_copy(dst_hbm.at[pl.ds(0, THALF)], dstidx, isem).wait()
    plsc.subcore_barrier()

    for ph in range(TCH // THALF):
        if ph > 0:
            pltpu.async_copy(
                src_hbm.at[pl.ds(wid * TCH + ph * THALF, THALF)], srcidx,
                isem)
            pltpu.async_copy(
                dst_hbm.at[pl.ds(wid * TCH + ph * THALF, THALF)], dstidx,
                isem)
            pltpu.make_async_copy(src_hbm.at[pl.ds(0, THALF)], srcidx,
                                  isem).wait()
            pltpu.make_async_copy(dst_hbm.at[pl.ds(0, THALF)], dstidx,
                                  isem).wait()

        for b in range(NBUF):
            pltpu.async_copy(hs_hbm.at[srcidx.at[b]], rows[b], gsem[b])

        @pl.loop(0, THALF, step=NBUF)
        def _(g):
            for b in range(NBUF):
                c = g + b
                pltpu.make_async_copy(hs_hbm.at[pl.ds(0, CH)], rows[b],
                                      gsem[b]).wait()
                pltpu.sync_copy(rows[b], acc.at[dstidx.at[c]], add=True)

                @pl.when(c + NBUF < THALF)
                def _():
                    pltpu.async_copy(hs_hbm.at[srcidx.at[c + NBUF]], rows[b],
                                     gsem[b])

    plsc.subcore_barrier()

    nq = jnp.where(sid < NQ % NS, NQ // NS + 1, NQ // NS)

    @pl.loop(0, nq)
    def _(k):
        r0 = (k * NS + sid) * ZROWS
        pltpu.sync_copy(acc.at[pl.ds(r0, ZROWS)],
                        out_hbm.at[pl.ds(cid * N + r0, ZROWS)])


@jax.jit
def _sc_agg(hs, src2d, dst2d):
    k = pl.kernel(
        _agg_body,
        out_type=jax.ShapeDtypeStruct((NC * N, H), jnp.float32),
        mesh=_sc_mesh(),
        scratch_types=[
            pltpu.VMEM((THALF, CH), jnp.int32),
            pltpu.VMEM((THALF, CH), jnp.int32),
            pltpu.VMEM((CH, H), jnp.float32),
            pltpu.VMEM((CH, H), jnp.float32),
            pltpu.VMEM((ZR, H), jnp.float32),
            pltpu.VMEM_SHARED((N, H), jnp.float32),
            pltpu.SemaphoreType.DMA,
            pltpu.SemaphoreType.DMA,
            pltpu.SemaphoreType.DMA,
        ],
    )
    return k(hs, src2d, dst2d)


def _emb_kernel(x_ref, wemb_ref, bemb_ref, wc_ref, o_ref):
    h = jnp.dot(x_ref[...], wemb_ref[...], precision=_HIGH,
                preferred_element_type=jnp.float32) + bemb_ref[...]
    o_ref[...] = jnp.dot(h, wc_ref[...], precision=_HIGH,
                         preferred_element_type=jnp.float32)


def _tc_emb(x, W_emb, b_emb, W_c0):
    return pl.pallas_call(
        _emb_kernel,
        out_shape=jax.ShapeDtypeStruct((N, H), jnp.float32),
    )(x, W_emb, b_emb.reshape(1, H), W_c0)


def _scale_kernel(degp_ref, hl_ref, hs_ref, dinv_ref):
    deg = degp_ref[0:N, 0:1] + degp_ref[N:2 * N, 0:1] + 1.0
    dinv = lax.rsqrt(deg)
    dinv_ref[...] = dinv
    hs_ref[0:N, :] = hl_ref[...] * dinv
    hs_ref[N:NXT, :] = jnp.zeros((NXT - N, H), jnp.float32)


def _tc_scale(degp, hl):
    return pl.pallas_call(
        _scale_kernel,
        out_shape=(jax.ShapeDtypeStruct((NXT, H), jnp.float32),
                   jax.ShapeDtypeStruct((N, 1), jnp.float32)),
    )(degp, hl)


def _mid_kernel(accp_ref, hs_ref, dinv_ref, b_ref, g_ref, bb_ref, w_ref,
                o_ref):
    dinv = dinv_ref[...]
    pre = (accp_ref[0:N, :] + accp_ref[N:2 * N, :] + hs_ref[0:N, :]) * dinv \
        + b_ref[...]
    mean = jnp.mean(pre, axis=0, keepdims=True)
    var = jnp.mean((pre - mean) ** 2, axis=0, keepdims=True)
    hn = (pre - mean) * lax.rsqrt(var + EPS) * g_ref[...] + bb_ref[...]
    h = jnp.maximum(hn, 0.0)
    o_ref[0:N, :] = jnp.dot(h, w_ref[...], precision=_HIGH,
                            preferred_element_type=jnp.float32) * dinv
    o_ref[N:NXT, :] = jnp.zeros((NXT - N, H), jnp.float32)


def _tc_mid(accp, hs, dinv, b, g, bb, W_next):
    return pl.pallas_call(
        _mid_kernel,
        out_shape=jax.ShapeDtypeStruct((NXT, H), jnp.float32),
    )(accp, hs, dinv, b.reshape(1, H), g.reshape(1, H), bb.reshape(1, H),
      W_next)


def _final_kernel(accp_ref, hs_ref, dinv_ref, b_ref, g_ref, bb_ref,
                  batch_ref, wh1_ref, bh1_ref, wh2_ref, bh2_ref, o_ref):
    dinv = dinv_ref[...]
    pre = (accp_ref[0:N, :] + accp_ref[N:2 * N, :] + hs_ref[0:N, :]) * dinv \
        + b_ref[...]
    mean = jnp.mean(pre, axis=0, keepdims=True)
    var = jnp.mean((pre - mean) ** 2, axis=0, keepdims=True)
    hn = (pre - mean) * lax.rsqrt(var + EPS) * g_ref[...] + bb_ref[...]
    h = jnp.maximum(hn, 0.0)
    seg = lax.broadcasted_iota(jnp.int32, (N, G), 1)
    onehot = (batch_ref[...] == seg).astype(jnp.float32)
    pooled = lax.dot_general(onehot, h, (((0,), (0,)), ((), ())),
                             precision=_HIGH,
                             preferred_element_type=jnp.float32)
    z = jnp.maximum(
        jnp.dot(pooled, wh1_ref[...], precision=_HIGH,
                preferred_element_type=jnp.float32) + bh1_ref[...], 0.0)
    o_ref[...] = jnp.dot(z, wh2_ref[...], precision=_HIGH,
                         preferred_element_type=jnp.float32) + bh2_ref[...]


def _tc_final(accp, hs, dinv, b, g, bb, batch2d, W_h1, b_h1, W_h2, b_h2):
    return pl.pallas_call(
        _final_kernel,
        out_shape=jax.ShapeDtypeStruct((G, OUT), jnp.float32),
    )(accp, hs, dinv, b.reshape(1, H), g.reshape(1, H), bb.reshape(1, H),
      batch2d, W_h1, b_h1.reshape(1, H // 2), W_h2, b_h2.reshape(1, OUT))


def kernel(x, edge_index, batch, W_emb, b_emb, W_c0, b_c0, bn_g0, bn_b0,
           W_c1, b_c1, bn_g1, bn_b1, W_c2, b_c2, bn_g2, bn_b2,
           W_h1, b_h1, W_h2, b_h2):
    src = edge_index[0]
    dst = edge_index[1]
    pad = EPAD - E
    src2d = jnp.concatenate(
        [src, N + (jnp.arange(pad, dtype=jnp.int32) % (NXT - N))]
    ).reshape(NCHUNK, CH)
    dst2d = jnp.concatenate(
        [dst, jnp.arange(pad, dtype=jnp.int32) % N]).reshape(NCHUNK, CH)
    batch2d = batch.reshape(N, 1)

    degp = _sc_deg(dst2d)
    hl0 = _tc_emb(x, W_emb, b_emb, W_c0)
    hs0, dinv = _tc_scale(degp, hl0)

    acc0 = _sc_agg(hs0, src2d, dst2d)
    hs1 = _tc_mid(acc0, hs0, dinv, b_c0, bn_g0, bn_b0, W_c1)
    acc1 = _sc_agg(hs1, src2d, dst2d)
    hs2 = _tc_mid(acc1, hs1, dinv, b_c1, bn_g1, bn_b1, W_c2)
    acc2 = _sc_agg(hs2, src2d, dst2d)

    return _tc_final(acc2, hs2, dinv, b_c2, bn_g2, bn_b2, batch2d,
                     W_h1, b_h1, W_h2, b_h2)
